# Initial kernel scaffold; baseline (speedup 1.0000x reference)
import jax
import jax.numpy as jnp
from jax.experimental import pallas as pl


def kernel(preds, low_feats, high_feats, SAMSegment, unlabeled_ROIs):
    raise NotImplementedError("write your pallas kernel here")


# speedup vs baseline: 172.6868x; 172.6868x over previous
import functools

import numpy as np
import jax
import jax.numpy as jnp
from jax import lax
from jax.experimental import pallas as pl
from jax.experimental.pallas import tpu as pltpu
from jax.experimental.pallas import tpu_sc as plsc

_SIGMA = 0.002
_WEIGHT = 0.4
_N = 4096
_M = 8064
_MP = 8192
_LVL_LEN = 4112
_MAXD_SLOT = 4104


def _resize_matrix(dst, src):
    scale = dst / src
    inv = 1.0 / scale
    out = np.zeros((dst, src), np.float32)
    for i in range(dst):
        center = (i + 0.5) * inv - 0.5
        js = np.arange(int(np.floor(center - inv)) - 1,
                       int(np.ceil(center + inv)) + 2)
        w = np.maximum(0.0, 1.0 - np.abs(js - center) * scale)
        m = (js >= 0) & (js < src)
        js, w = js[m], w[m]
        out[i, js] = w / w.sum()
    return out


_A64 = _resize_matrix(64, 256)


def _edge_key_mat(x):
    dh = ((x[:, :, :63] - x[:, :, 1:64]) ** 2).sum(0)
    dv = ((x[:, :63, :] - x[:, 1:64, :]) ** 2).sum(0)
    inf_col = jnp.full((64, 1), jnp.inf, jnp.float32)
    inf_row = jnp.full((1, 64), jnp.inf, jnp.float32)
    left = jnp.concatenate([dh, inf_col], axis=1)
    right = jnp.concatenate([dv, inf_row], axis=0)
    return jnp.concatenate([left, right], axis=1)


def _sort_kernel_body(low_ref, high_ref, a_ref, sw_ref, si_ref):
    A = a_ref[:]
    keys = []
    for b in range(2):
        lows = []
        for c in range(3):
            X = low_ref[b, c]
            Y = jnp.dot(jnp.dot(A, X, preferred_element_type=jnp.float32),
                        A.T, preferred_element_type=jnp.float32)
            lows.append(Y)
        low = jnp.stack(lows, 0)
        keys.append(_edge_key_mat(low))
        keys.append(_edge_key_mat(high_ref[b]))
    K = jnp.stack(keys, 0)

    row = lax.broadcasted_iota(jnp.int32, (64, 128), 0)
    col = lax.broadcasted_iota(jnp.int32, (64, 128), 1)
    lin = row * 128 + col
    idx = jnp.where(col < 63, row * 63 + col, 4032 + row * 64 + (col - 64))
    pad = (col == 63) | ((col >= 64) & (row == 63))
    idx = jnp.where(pad, _MP + lin, idx)
    I = jnp.broadcast_to(idx[None], (4, 64, 128)).astype(jnp.int32)
    lin3 = jnp.broadcast_to(lin[None], (4, 64, 128))

    for k in range(13):
        for j in range(k, -1, -1):
            d = 1 << j
            if d < 128:
                ax, sh, size = 2, d, 128
            else:
                ax, sh, size = 1, d >> 7, 64
            low_half = (lin3 & d) == 0
            pK = jnp.where(low_half, pltpu.roll(K, size - sh, ax),
                           pltpu.roll(K, sh, ax))
            pI = jnp.where(low_half, pltpu.roll(I, size - sh, ax),
                           pltpu.roll(I, sh, ax))
            asc = (lin3 & (1 << (k + 1))) == 0
            partner_less = (pK < K) | ((pK == K) & (pI < I))
            take = asc ^ low_half ^ partner_less
            K = jnp.where(take, pK, K)
            I = jnp.where(take, pI, I)

    sw_ref[:] = K
    si_ref[:] = I


def _tc_sort(low_feats, high_feats):
    A = jnp.asarray(_A64)
    sw, si = pl.pallas_call(
        _sort_kernel_body,
        out_shape=(
            jax.ShapeDtypeStruct((4, 64, 128), jnp.float32),
            jax.ShapeDtypeStruct((4, 64, 128), jnp.int32),
        ),
    )(low_feats, high_feats, A)
    return sw.reshape(4 * _MP), si.reshape(4 * _MP)


_IOTA = lambda: lax.iota(jnp.int32, 16)


def _vg(ref, idx, mask=None):
    return plsc.load_gather(ref, [idx], mask=mask)


def _splat(x, dtype=jnp.int32):
    return jnp.full((16,), x, dtype)


def _lane0():
    return _IOTA() == 0


def _sstore(ref, idx_vec, val_vec):
    plsc.store_scatter(ref, [idx_vec], val_vec, mask=_lane0())


def _fill(ref, n_chunks, value, dtype):
    def body(i, _):
        idx = i * 16 + _IOTA()
        plsc.store_scatter(ref, [idx], _splat(value, dtype))
        return 0
    lax.fori_loop(0, n_chunks, body, 0)


def _uf_find(par_ref, x0):
    p0 = _vg(par_ref, x0)

    def cond(c):
        x, p = c
        return jnp.any(p != x)

    def body(c):
        x, p = c
        g = _vg(par_ref, p)
        _sstore(par_ref, x, g)
        return g, _vg(par_ref, g)

    x, _ = lax.while_loop(cond, body, (x0, p0))
    return x


def _build_tree(sw_ref, sidx_ref, par_ref, adjm_ref, wes_ref,
                parent_ref, d2w_ref, ordq_ref, lvl_ref):
    iota = _IOTA()

    def initb(i, _):
        idx = i * 16 + iota
        plsc.store_scatter(par_ref, [idx], idx)
        plsc.store_scatter(adjm_ref, [idx], _splat(0))
        plsc.store_scatter(parent_ref, [idx], _splat(-1))
        plsc.store_scatter(d2w_ref, [idx], _splat(0.0, jnp.float32))
        return 0
    lax.fori_loop(0, _N // 16, initb, 0)

    def kr_cond(c):
        e, cnt = c
        return jnp.any((e < _M) & (cnt < _N - 1))

    def kr_body(c):
        e, cnt = c
        eidx = _vg(sidx_ref, e)
        horiz = eidx < 4032
        r = eidx // 63
        cc = eidx - r * 63
        a = jnp.where(horiz, r * 64 + cc, eidx - 4032)
        b = jnp.where(horiz, a + 1, a + 64)
        ra = _uf_find(par_ref, a)
        rb = _uf_find(par_ref, b)
        take = ra != rb

        @pl.when(jnp.any(take))
        def _():
            _sstore(par_ref, ra, rb)
            w_e = _vg(sw_ref, e)
            bit_a = jnp.where(horiz, 1, 4)
            bit_b = jnp.where(horiz, 2, 8)
            plsc.addupdate_scatter(adjm_ref, [a], bit_a, mask=_lane0())
            plsc.addupdate_scatter(adjm_ref, [b], bit_b, mask=_lane0())
            wpos = jnp.where(horiz, a, a + _N)
            plsc.store_scatter(wes_ref, [wpos], w_e, mask=_lane0())

        return e + 1, cnt + jnp.where(take, 1, 0)

    lax.while_loop(kr_cond, kr_body, (_splat(0), _splat(0)))

    _sstore(parent_ref, _splat(0), _splat(0))
    _sstore(ordq_ref, _splat(0), _splat(0))
    _sstore(lvl_ref, _splat(0), _splat(0))
    _sstore(lvl_ref, _splat(1), _splat(1))

    def lvl_cond(c):
        s, t, l = c
        return jnp.any((t < _N) & (l < _N + 2))

    def lvl_body(c):
        s, t, l = c

        def ch_cond(cc):
            q, t2 = cc
            return jnp.any(q < t)

        def ch_body(cc):
            q, t2 = cc
            m_in = iota < (t - q)
            v = _vg(ordq_ref, jnp.where(m_in, q + iota, 0))
            am = _vg(adjm_ref, v)
            for bit, dd in ((1, 1), (2, -1), (4, 64), (8, -64)):
                hasd = m_in & ((am & bit) != 0)
                u = v + dd
                uc = jnp.minimum(jnp.maximum(u, 0), _N - 1)
                pu = _vg(parent_ref, uc, mask=hasd)
                newm = hasd & (pu < 0)
                plsc.store_scatter(parent_ref, [uc], v, mask=newm)
                if bit == 1:
                    wpos = v
                elif bit == 2:
                    wpos = uc
                elif bit == 4:
                    wpos = v + _N
                else:
                    wpos = uc + _N
                wv = _vg(wes_ref, wpos, mask=newm)
                plsc.store_scatter(d2w_ref, [uc], wv, mask=newm)
                kc = plsc.cumsum(newm.astype(jnp.int32))
                pos = t2 + kc - 1
                plsc.store_scatter(ordq_ref, [pos], uc, mask=newm)
                t2 = t2 + plsc.all_reduce_population_count(newm)
            return q + 16, t2

        q, t2 = lax.while_loop(ch_cond, ch_body, (s, t))
        _sstore(lvl_ref, l + 1, t2)
        return t, t2, l + 1

    s, t, l = lax.while_loop(lvl_cond, lvl_body,
                             (_splat(0), _splat(1), _splat(1)))
    _sstore(lvl_ref, _splat(_MAXD_SLOT), l - 1)

    def wgtb(i, _):
        idx = i * 16 + iota
        d2v = _vg(d2w_ref, idx)
        plsc.store_scatter(d2w_ref, [idx], jnp.exp(-_SIGMA * d2v))
        return 0
    lax.fori_loop(0, _N // 16, wgtb, 0)
    _sstore(d2w_ref, _splat(0), _splat(0.0, jnp.float32))


def _run_filter(parent_ref, wgt_ref, ordq_ref, lvl_ref, feat_ref,
                aggf_ref, aggn_ref, df_ref, dn_ref):
    iota = _IOTA()
    maxd = _vg(lvl_ref, _splat(_MAXD_SLOT))

    def initb(i, _):
        idx = i * 16 + iota
        plsc.store_scatter(aggf_ref, [idx], _vg(feat_ref, idx))
        plsc.store_scatter(aggn_ref, [idx], _splat(1.0, jnp.float32))
        return 0
    lax.fori_loop(0, _N // 16, initb, 0)

    def up_cond(l):
        return jnp.any(l >= 1)

    def up_body(l):
        s = _vg(lvl_ref, l)
        e = _vg(lvl_ref, l + 1)

        def cc(q):
            return jnp.any(q < e)

        def cb(q):
            m = iota < (e - q)
            v = jnp.where(m, _vg(ordq_ref, q + iota), 0)
            p = _vg(parent_ref, v)
            w = _vg(wgt_ref, v)
            fa = _vg(aggf_ref, v)
            fn = _vg(aggn_ref, v)
            plsc.addupdate_scatter(aggf_ref, [p], w * fa, mask=m)
            plsc.addupdate_scatter(aggn_ref, [p], w * fn, mask=m)
            return q + 16

        lax.while_loop(cc, cb, s)
        return l - 1

    lax.while_loop(up_cond, up_body, maxd)

    def cpb(i, _):
        idx = i * 16 + iota
        plsc.store_scatter(df_ref, [idx], _vg(aggf_ref, idx))
        plsc.store_scatter(dn_ref, [idx], _vg(aggn_ref, idx))
        return 0
    lax.fori_loop(0, _N // 16, cpb, 0)

    def dn_cond(l):
        return jnp.any(l <= maxd)

    def dn_body(l):
        s = _vg(lvl_ref, l)
        e = _vg(lvl_ref, l + 1)

        def cc(q):
            return jnp.any(q < e)

        def cb(q):
            m = iota < (e - q)
            v = jnp.where(m, _vg(ordq_ref, q + iota), 0)
            p = _vg(parent_ref, v)
            w = _vg(wgt_ref, v)
            fa = _vg(aggf_ref, v)
            fn = _vg(aggn_ref, v)
            nf = fa + w * (_vg(df_ref, p) - w * fa)
            nn = fn + w * (_vg(dn_ref, p) - w * fn)
            plsc.store_scatter(df_ref, [v], nf, mask=m)
            plsc.store_scatter(dn_ref, [v], nn, mask=m)
            return q + 16

        lax.while_loop(cc, cb, s)
        return l + 1

    lax.while_loop(dn_cond, dn_body, _splat(1))

    def outb(i, _):
        idx = i * 16 + iota
        plsc.store_scatter(feat_ref, [idx],
                           _vg(df_ref, idx) / _vg(dn_ref, idx))
        return 0
    lax.fori_loop(0, _N // 16, outb, 0)


def _sc_body(sw_hbm, sidx_hbm, preds_hbm, out_hbm,
             sw_v, sidx_v, par_v, adjm_v, wes_v, parent_v, d2w_v,
             ordq_v, lvl_v, aggf_v, aggn_v, df_v, dn_v, feat_v,
             parent_sh, wgt_sh, ordq_sh, lvl_sh):
    cid = lax.axis_index("c")
    sid = lax.axis_index("s")
    t = 2 * cid + sid

    @pl.when(sid < 2)
    def _():
        pltpu.sync_copy(sw_hbm.at[pl.ds(t * _MP, _MP)], sw_v)
        pltpu.sync_copy(sidx_hbm.at[pl.ds(t * _MP, _MP)], sidx_v)
        _build_tree(sw_v, sidx_v, par_v, adjm_v, wes_v,
                    parent_v, d2w_v, ordq_v, lvl_v)

    @pl.when(sid == 1)
    def _():
        pltpu.sync_copy(parent_v, parent_sh)
        pltpu.sync_copy(d2w_v, wgt_sh)
        pltpu.sync_copy(ordq_v, ordq_sh)
        pltpu.sync_copy(lvl_v, lvl_sh)

    @pl.when(sid == 0)
    def _():
        pltpu.sync_copy(preds_hbm.at[pl.ds(cid * _N, _N)], feat_v)
        _run_filter(parent_v, d2w_v, ordq_v, lvl_v, feat_v,
                    aggf_v, aggn_v, df_v, dn_v)

    plsc.subcore_barrier()

    @pl.when(sid == 0)
    def _():
        pltpu.sync_copy(parent_sh, parent_v)
        pltpu.sync_copy(wgt_sh, d2w_v)
        pltpu.sync_copy(ordq_sh, ordq_v)
        pltpu.sync_copy(lvl_sh, lvl_v)
        _run_filter(parent_v, d2w_v, ordq_v, lvl_v, feat_v,
                    aggf_v, aggn_v, df_v, dn_v)
        pltpu.sync_copy(feat_v, out_hbm.at[pl.ds(cid * _N, _N)])


def _sc_filter(sw, sidx, preds_flat):
    mesh = plsc.VectorSubcoreMesh(core_axis_name="c", subcore_axis_name="s",
                                  num_cores=2, num_subcores=16)
    f = pl.kernel(
        _sc_body,
        out_type=jax.ShapeDtypeStruct((2 * _N,), jnp.float32),
        mesh=mesh,
        compiler_params=pltpu.CompilerParams(needs_layout_passes=False),
        scratch_types=[
            pltpu.VMEM((_MP,), jnp.float32),
            pltpu.VMEM((_MP,), jnp.int32),
            pltpu.VMEM((_N,), jnp.int32),
            pltpu.VMEM((_N,), jnp.int32),
            pltpu.VMEM((2 * _N,), jnp.float32),
            pltpu.VMEM((_N,), jnp.int32),
            pltpu.VMEM((_N,), jnp.float32),
            pltpu.VMEM((_LVL_LEN,), jnp.int32),
            pltpu.VMEM((_LVL_LEN,), jnp.int32),
            pltpu.VMEM((_N,), jnp.float32),
            pltpu.VMEM((_N,), jnp.float32),
            pltpu.VMEM((_N,), jnp.float32),
            pltpu.VMEM((_N,), jnp.float32),
            pltpu.VMEM((_N,), jnp.float32),
            pltpu.VMEM_SHARED((_N,), jnp.int32),
            pltpu.VMEM_SHARED((_N,), jnp.float32),
            pltpu.VMEM_SHARED((_LVL_LEN,), jnp.int32),
            pltpu.VMEM_SHARED((_LVL_LEN,), jnp.int32),
        ],
    )
    return f(sw, sidx, preds_flat)


def _loss_body(preds_ref, sam_ref, unl_ref, as_ref, out_ref):
    sam = sam_ref[:]
    sam_bg = 1.0 - 1.0 / (1.0 + jnp.exp(-sam))
    as_c = (sam_bg + as_ref[:]) * 0.5
    unl = unl_ref[:]
    nc = jnp.sum(unl)
    loss = jnp.sum(unl * jnp.abs(preds_ref[:] - as_c))
    loss = jnp.where(nc > 0, loss / nc, loss)
    out_ref[:, :] = jnp.full((1, 1), _WEIGHT * loss, jnp.float32)


def _tc_loss(preds_f, sam_f, unl_f, as_f):
    out = pl.pallas_call(
        _loss_body,
        out_shape=jax.ShapeDtypeStruct((1, 1), jnp.float32),
    )(preds_f, sam_f, unl_f, as_f)
    return out.reshape(())


def kernel(preds, low_feats, high_feats, SAMSegment, unlabeled_ROIs):
    b = preds.shape[0]
    n = _N
    sw, sidx = _tc_sort(low_feats, high_feats)
    as_flat = _sc_filter(sw, sidx, preds.reshape(b * n))
    return _tc_loss(preds.reshape(b, n), SAMSegment.reshape(b, n),
                    unlabeled_ROIs.reshape(b, n), as_flat.reshape(b, n))


# speedup vs baseline: 410.3929x; 2.3765x over previous
import functools

import numpy as np
import jax
import jax.numpy as jnp
from jax import lax
from jax.experimental import pallas as pl
from jax.experimental.pallas import tpu as pltpu
from jax.experimental.pallas import tpu_sc as plsc

_SIGMA = 0.002
_WEIGHT = 0.4
_N = 4096
_M = 8064
_MP = 8192
_LVL_LEN = 4112
_MAXD_SLOT = 4104


def _resize_matrix(dst, src):
    scale = dst / src
    inv = 1.0 / scale
    out = np.zeros((dst, src), np.float32)
    for i in range(dst):
        center = (i + 0.5) * inv - 0.5
        js = np.arange(int(np.floor(center - inv)) - 1,
                       int(np.ceil(center + inv)) + 2)
        w = np.maximum(0.0, 1.0 - np.abs(js - center) * scale)
        m = (js >= 0) & (js < src)
        js, w = js[m], w[m]
        out[i, js] = w / w.sum()
    return out


_A64 = _resize_matrix(64, 256)


def _edge_key_mat(x):
    dh = ((x[:, :, :63] - x[:, :, 1:64]) ** 2).sum(0)
    dv = ((x[:, :63, :] - x[:, 1:64, :]) ** 2).sum(0)
    inf_col = jnp.full((64, 1), jnp.inf, jnp.float32)
    inf_row = jnp.full((1, 64), jnp.inf, jnp.float32)
    left = jnp.concatenate([dh, inf_col], axis=1)
    right = jnp.concatenate([dv, inf_row], axis=0)
    return jnp.concatenate([left, right], axis=1)


def _sort_kernel_body(low_ref, high_ref, a_ref, sw_ref, si_ref):
    A = a_ref[:]
    keys = []
    for b in range(2):
        lows = []
        for c in range(3):
            X = low_ref[b, c]
            Y = jnp.dot(jnp.dot(A, X, preferred_element_type=jnp.float32),
                        A.T, preferred_element_type=jnp.float32)
            lows.append(Y)
        low = jnp.stack(lows, 0)
        keys.append(_edge_key_mat(low))
        keys.append(_edge_key_mat(high_ref[b]))
    K = jnp.stack(keys, 0)

    row = lax.broadcasted_iota(jnp.int32, (64, 128), 0)
    col = lax.broadcasted_iota(jnp.int32, (64, 128), 1)
    lin = row * 128 + col
    idx = jnp.where(col < 63, row * 63 + col, 4032 + row * 64 + (col - 64))
    pad = (col == 63) | ((col >= 64) & (row == 63))
    idx = jnp.where(pad, _MP + lin, idx)
    I = jnp.broadcast_to(idx[None], (4, 64, 128)).astype(jnp.int32)
    lin3 = jnp.broadcast_to(lin[None], (4, 64, 128))

    for k in range(13):
        for j in range(k, -1, -1):
            d = 1 << j
            if d < 128:
                ax, sh, size = 2, d, 128
            else:
                ax, sh, size = 1, d >> 7, 64
            low_half = (lin3 & d) == 0
            pK = jnp.where(low_half, pltpu.roll(K, size - sh, ax),
                           pltpu.roll(K, sh, ax))
            pI = jnp.where(low_half, pltpu.roll(I, size - sh, ax),
                           pltpu.roll(I, sh, ax))
            asc = (lin3 & (1 << (k + 1))) == 0
            partner_less = (pK < K) | ((pK == K) & (pI < I))
            take = asc ^ low_half ^ partner_less
            K = jnp.where(take, pK, K)
            I = jnp.where(take, pI, I)

    sw_ref[:] = K
    si_ref[:] = I


def _tc_sort(low_feats, high_feats):
    A = jnp.asarray(_A64)
    sw, si = pl.pallas_call(
        _sort_kernel_body,
        out_shape=(
            jax.ShapeDtypeStruct((4, 64, 128), jnp.float32),
            jax.ShapeDtypeStruct((4, 64, 128), jnp.int32),
        ),
    )(low_feats, high_feats, A)
    return sw.reshape(4 * _MP), si.reshape(4 * _MP)


_IOTA = lambda: lax.iota(jnp.int32, 16)


def _vg(ref, idx, mask=None):
    return plsc.load_gather(ref, [idx], mask=mask)


def _splat(x, dtype=jnp.int32):
    return jnp.full((16,), x, dtype)


def _lane0():
    return _IOTA() == 0


def _sstore(ref, idx_vec, val_vec):
    plsc.store_scatter(ref, [idx_vec], val_vec, mask=_lane0())


def _fill(ref, n_chunks, value, dtype):
    def body(i, _):
        idx = i * 16 + _IOTA()
        plsc.store_scatter(ref, [idx], _splat(value, dtype))
        return 0
    lax.fori_loop(0, n_chunks, body, 0)


def _uf_find(par_ref, x0):
    p0 = _vg(par_ref, x0)

    def cond(c):
        x, p = c
        return jnp.any(p != x)

    def body(c):
        x, p = c
        g = _vg(par_ref, p)
        _sstore(par_ref, x, g)
        return g, _vg(par_ref, g)

    x, _ = lax.while_loop(cond, body, (x0, p0))
    return x


def _build_tree(sw_ref, sidx_ref, par_ref, adjm_ref, wes_ref,
                parent_ref, d2w_ref, ordq_ref, lvl_ref):
    iota = _IOTA()

    def initb(i, _):
        idx = i * 16 + iota
        plsc.store_scatter(par_ref, [idx], idx)
        plsc.store_scatter(adjm_ref, [idx], _splat(0))
        plsc.store_scatter(parent_ref, [idx], _splat(-1))
        plsc.store_scatter(d2w_ref, [idx], _splat(0.0, jnp.float32))
        return 0
    lax.fori_loop(0, _N // 16, initb, 0)

    def bc(vec, j):
        return vec.at[_splat(j)].get(mode="promise_in_bounds")

    def find_loop(x, p):
        def cond(c):
            x, p = c
            return jnp.any(p != x)

        def body(c):
            x, p = c
            live = p != x
            g = _vg(par_ref, p)
            plsc.store_scatter(par_ref, [x], g, mask=live)
            x2 = jnp.where(live, g, x)
            return x2, _vg(par_ref, x2)

        return lax.while_loop(cond, body, (x, p))[0]

    def kr_cond(c):
        e, cnt = c
        return jnp.any((e < _M) & (cnt < _N - 1))

    def kr_body(c):
        e, cnt = c
        lane8 = iota & 7
        eidx8 = _vg(sidx_ref, e + lane8)
        w8 = _vg(sw_ref, e + lane8)
        horiz8 = eidx8 < 4032
        r8 = eidx8 // 63
        c8 = eidx8 - r8 * 63
        a8 = jnp.where(horiz8, r8 * 64 + c8, eidx8 - 4032)
        b8 = jnp.where(horiz8, a8 + 1, a8 + 64)
        x0 = jnp.where(iota < 8, a8, b8)
        roots = find_loop(x0, _vg(par_ref, x0))

        for j in range(8):
            xj = jnp.where(iota < 8, bc(roots, j), bc(roots, j + 8))
            xj = find_loop(xj, _vg(par_ref, xj))
            ra = bc(xj, 0)
            rb = bc(xj, 8)
            take = ra != rb

            @pl.when(jnp.any(take))
            def _(j=j, ra=ra, rb=rb):
                eidx = bc(eidx8, j)
                horiz = eidx < 4032
                a = bc(a8, j)
                b = bc(b8, j)
                _sstore(par_ref, ra, rb)
                bit_a = jnp.where(horiz, 1, 4)
                bit_b = jnp.where(horiz, 2, 8)
                plsc.addupdate_scatter(adjm_ref, [a], bit_a, mask=_lane0())
                plsc.addupdate_scatter(adjm_ref, [b], bit_b, mask=_lane0())
                wpos = jnp.where(horiz, a, a + _N)
                plsc.store_scatter(wes_ref, [wpos], bc(w8, j), mask=_lane0())

            cnt = cnt + jnp.where(take, 1, 0)

        return e + 8, cnt

    lax.while_loop(kr_cond, kr_body, (_splat(0), _splat(0)))

    _sstore(parent_ref, _splat(0), _splat(0))
    _sstore(ordq_ref, _splat(0), _splat(0))
    _sstore(lvl_ref, _splat(0), _splat(0))
    _sstore(lvl_ref, _splat(1), _splat(1))

    def lvl_cond(c):
        s, t, l = c
        return jnp.any((t < _N) & (l < _N + 2))

    def lvl_body(c):
        s, t, l = c

        def ch_cond(cc):
            q, t2 = cc
            return jnp.any(q < t)

        def ch_body(cc):
            q, t2 = cc
            m_in = iota < (t - q)
            v = _vg(ordq_ref, jnp.where(m_in, q + iota, 0))
            am = _vg(adjm_ref, v)
            for bit, dd in ((1, 1), (2, -1), (4, 64), (8, -64)):
                hasd = m_in & ((am & bit) != 0)
                u = v + dd
                uc = jnp.minimum(jnp.maximum(u, 0), _N - 1)
                pu = _vg(parent_ref, uc, mask=hasd)
                newm = hasd & (pu < 0)
                plsc.store_scatter(parent_ref, [uc], v, mask=newm)
                if bit == 1:
                    wpos = v
                elif bit == 2:
                    wpos = uc
                elif bit == 4:
                    wpos = v + _N
                else:
                    wpos = uc + _N
                wv = _vg(wes_ref, wpos, mask=newm)
                plsc.store_scatter(d2w_ref, [uc], wv, mask=newm)
                kc = plsc.cumsum(newm.astype(jnp.int32))
                pos = t2 + kc - 1
                plsc.store_scatter(ordq_ref, [pos], uc, mask=newm)
                t2 = t2 + plsc.all_reduce_population_count(newm)
            return q + 16, t2

        q, t2 = lax.while_loop(ch_cond, ch_body, (s, t))
        _sstore(lvl_ref, l + 1, t2)
        return t, t2, l + 1

    s, t, l = lax.while_loop(lvl_cond, lvl_body,
                             (_splat(0), _splat(1), _splat(1)))
    _sstore(lvl_ref, _splat(_MAXD_SLOT), l - 1)

    def wgtb(i, _):
        idx = i * 16 + iota
        d2v = _vg(d2w_ref, idx)
        plsc.store_scatter(d2w_ref, [idx], jnp.exp(-_SIGMA * d2v))
        return 0
    lax.fori_loop(0, _N // 16, wgtb, 0)
    _sstore(d2w_ref, _splat(0), _splat(0.0, jnp.float32))


def _run_filter(parent_ref, wgt_ref, ordq_ref, lvl_ref, feat_ref,
                aggf_ref, aggn_ref, df_ref, dn_ref):
    iota = _IOTA()
    maxd = _vg(lvl_ref, _splat(_MAXD_SLOT))

    def initb(i, _):
        idx = i * 16 + iota
        plsc.store_scatter(aggf_ref, [idx], _vg(feat_ref, idx))
        plsc.store_scatter(aggn_ref, [idx], _splat(1.0, jnp.float32))
        return 0
    lax.fori_loop(0, _N // 16, initb, 0)

    def up_cond(l):
        return jnp.any(l >= 1)

    def up_body(l):
        s = _vg(lvl_ref, l)
        e = _vg(lvl_ref, l + 1)

        def cc(q):
            return jnp.any(q < e)

        def cb(q):
            m = iota < (e - q)
            v = jnp.where(m, _vg(ordq_ref, q + iota), 0)
            p = _vg(parent_ref, v)
            w = _vg(wgt_ref, v)
            fa = _vg(aggf_ref, v)
            fn = _vg(aggn_ref, v)
            plsc.addupdate_scatter(aggf_ref, [p], w * fa, mask=m)
            plsc.addupdate_scatter(aggn_ref, [p], w * fn, mask=m)
            return q + 16

        lax.while_loop(cc, cb, s)
        return l - 1

    lax.while_loop(up_cond, up_body, maxd)

    def cpb(i, _):
        idx = i * 16 + iota
        plsc.store_scatter(df_ref, [idx], _vg(aggf_ref, idx))
        plsc.store_scatter(dn_ref, [idx], _vg(aggn_ref, idx))
        return 0
    lax.fori_loop(0, _N // 16, cpb, 0)

    def dn_cond(l):
        return jnp.any(l <= maxd)

    def dn_body(l):
        s = _vg(lvl_ref, l)
        e = _vg(lvl_ref, l + 1)

        def cc(q):
            return jnp.any(q < e)

        def cb(q):
            m = iota < (e - q)
            v = jnp.where(m, _vg(ordq_ref, q + iota), 0)
            p = _vg(parent_ref, v)
            w = _vg(wgt_ref, v)
            fa = _vg(aggf_ref, v)
            fn = _vg(aggn_ref, v)
            nf = fa + w * (_vg(df_ref, p) - w * fa)
            nn = fn + w * (_vg(dn_ref, p) - w * fn)
            plsc.store_scatter(df_ref, [v], nf, mask=m)
            plsc.store_scatter(dn_ref, [v], nn, mask=m)
            return q + 16

        lax.while_loop(cc, cb, s)
        return l + 1

    lax.while_loop(dn_cond, dn_body, _splat(1))

    def outb(i, _):
        idx = i * 16 + iota
        plsc.store_scatter(feat_ref, [idx],
                           _vg(df_ref, idx) / _vg(dn_ref, idx))
        return 0
    lax.fori_loop(0, _N // 16, outb, 0)


def _sc_body(sw_hbm, sidx_hbm, preds_hbm, out_hbm,
             sw_v, sidx_v, par_v, adjm_v, wes_v, parent_v, d2w_v,
             ordq_v, lvl_v, aggf_v, aggn_v, df_v, dn_v, feat_v,
             parent_sh, wgt_sh, ordq_sh, lvl_sh):
    cid = lax.axis_index("c")
    sid = lax.axis_index("s")
    t = 2 * cid + sid

    @pl.when(sid < 2)
    def _():
        pltpu.sync_copy(sw_hbm.at[pl.ds(t * _MP, _MP)], sw_v)
        pltpu.sync_copy(sidx_hbm.at[pl.ds(t * _MP, _MP)], sidx_v)
        _build_tree(sw_v, sidx_v, par_v, adjm_v, wes_v,
                    parent_v, d2w_v, ordq_v, lvl_v)

    @pl.when(sid == 1)
    def _():
        pltpu.sync_copy(parent_v, parent_sh)
        pltpu.sync_copy(d2w_v, wgt_sh)
        pltpu.sync_copy(ordq_v, ordq_sh)
        pltpu.sync_copy(lvl_v, lvl_sh)

    @pl.when(sid == 0)
    def _():
        pltpu.sync_copy(preds_hbm.at[pl.ds(cid * _N, _N)], feat_v)
        _run_filter(parent_v, d2w_v, ordq_v, lvl_v, feat_v,
                    aggf_v, aggn_v, df_v, dn_v)

    plsc.subcore_barrier()

    @pl.when(sid == 0)
    def _():
        pltpu.sync_copy(parent_sh, parent_v)
        pltpu.sync_copy(wgt_sh, d2w_v)
        pltpu.sync_copy(ordq_sh, ordq_v)
        pltpu.sync_copy(lvl_sh, lvl_v)
        _run_filter(parent_v, d2w_v, ordq_v, lvl_v, feat_v,
                    aggf_v, aggn_v, df_v, dn_v)
        pltpu.sync_copy(feat_v, out_hbm.at[pl.ds(cid * _N, _N)])


def _sc_filter(sw, sidx, preds_flat):
    mesh = plsc.VectorSubcoreMesh(core_axis_name="c", subcore_axis_name="s",
                                  num_cores=2, num_subcores=16)
    f = pl.kernel(
        _sc_body,
        out_type=jax.ShapeDtypeStruct((2 * _N,), jnp.float32),
        mesh=mesh,
        compiler_params=pltpu.CompilerParams(needs_layout_passes=False),
        scratch_types=[
            pltpu.VMEM((_MP,), jnp.float32),
            pltpu.VMEM((_MP,), jnp.int32),
            pltpu.VMEM((_N,), jnp.int32),
            pltpu.VMEM((_N,), jnp.int32),
            pltpu.VMEM((2 * _N,), jnp.float32),
            pltpu.VMEM((_N,), jnp.int32),
            pltpu.VMEM((_N,), jnp.float32),
            pltpu.VMEM((_LVL_LEN,), jnp.int32),
            pltpu.VMEM((_LVL_LEN,), jnp.int32),
            pltpu.VMEM((_N,), jnp.float32),
            pltpu.VMEM((_N,), jnp.float32),
            pltpu.VMEM((_N,), jnp.float32),
            pltpu.VMEM((_N,), jnp.float32),
            pltpu.VMEM((_N,), jnp.float32),
            pltpu.VMEM_SHARED((_N,), jnp.int32),
            pltpu.VMEM_SHARED((_N,), jnp.float32),
            pltpu.VMEM_SHARED((_LVL_LEN,), jnp.int32),
            pltpu.VMEM_SHARED((_LVL_LEN,), jnp.int32),
        ],
    )
    return f(sw, sidx, preds_flat)


def _loss_body(preds_ref, sam_ref, unl_ref, as_ref, out_ref):
    sam = sam_ref[:]
    sam_bg = 1.0 - 1.0 / (1.0 + jnp.exp(-sam))
    as_c = (sam_bg + as_ref[:]) * 0.5
    unl = unl_ref[:]
    nc = jnp.sum(unl)
    loss = jnp.sum(unl * jnp.abs(preds_ref[:] - as_c))
    loss = jnp.where(nc > 0, loss / nc, loss)
    out_ref[:, :] = jnp.full((1, 1), _WEIGHT * loss, jnp.float32)


def _tc_loss(preds_f, sam_f, unl_f, as_f):
    out = pl.pallas_call(
        _loss_body,
        out_shape=jax.ShapeDtypeStruct((1, 1), jnp.float32),
    )(preds_f, sam_f, unl_f, as_f)
    return out.reshape(())


def kernel(preds, low_feats, high_feats, SAMSegment, unlabeled_ROIs):
    b = preds.shape[0]
    n = _N
    sw, sidx = _tc_sort(low_feats, high_feats)
    as_flat = _sc_filter(sw, sidx, preds.reshape(b * n))
    return _tc_loss(preds.reshape(b, n), SAMSegment.reshape(b, n),
                    unlabeled_ROIs.reshape(b, n), as_flat.reshape(b, n))


# speedup vs baseline: 523.6854x; 1.2761x over previous
import functools

import numpy as np
import jax
import jax.numpy as jnp
from jax import lax
from jax.experimental import pallas as pl
from jax.experimental.pallas import tpu as pltpu
from jax.experimental.pallas import tpu_sc as plsc

_SIGMA = 0.002
_WEIGHT = 0.4
_N = 4096
_M = 8064
_MP = 8192
_LVL_LEN = 4112
_MAXD_SLOT = 4104


def _resize_matrix(dst, src):
    scale = dst / src
    inv = 1.0 / scale
    out = np.zeros((dst, src), np.float32)
    for i in range(dst):
        center = (i + 0.5) * inv - 0.5
        js = np.arange(int(np.floor(center - inv)) - 1,
                       int(np.ceil(center + inv)) + 2)
        w = np.maximum(0.0, 1.0 - np.abs(js - center) * scale)
        m = (js >= 0) & (js < src)
        js, w = js[m], w[m]
        out[i, js] = w / w.sum()
    return out


_A64 = _resize_matrix(64, 256)


def _edge_key_mat(x):
    dh = ((x[:, :, :63] - x[:, :, 1:64]) ** 2).sum(0)
    dv = ((x[:, :63, :] - x[:, 1:64, :]) ** 2).sum(0)

    r_h = lax.broadcasted_iota(jnp.int32, (64, 63), 0)
    c_h = lax.broadcasted_iota(jnp.int32, (64, 63), 1)
    idx_h = r_h * 63 + c_h
    r_v = lax.broadcasted_iota(jnp.int32, (63, 64), 0)
    c_v = lax.broadcasted_iota(jnp.int32, (63, 64), 1)
    idx_v = 4032 + r_v * 64 + c_v

    def lexmax(w1, i1, w2, i2):
        t = (w1 > w2) | ((w1 == w2) & (i1 > i2))
        return jnp.where(t, w1, w2), jnp.where(t, i1, i2)

    top_w, top_i = dh[:63, :], idx_h[:63, :]
    bot_w, bot_i = dh[1:, :], idx_h[1:, :]
    lef_w, lef_i = dv[:, :63], idx_v[:, :63]
    rig_w, rig_i = dv[:, 1:], idx_v[:, 1:]
    mw, mi = lexmax(*lexmax(top_w, top_i, bot_w, bot_i),
                    *lexmax(lef_w, lef_i, rig_w, rig_i))
    one = jnp.float32(1.0)
    zero = jnp.float32(0.0)
    is_top = jnp.where(mi == top_i, one, zero)
    is_bot = jnp.where(mi == bot_i, one, zero)
    is_lef = jnp.where(mi == lef_i, one, zero)
    is_rig = jnp.where(mi == rig_i, one, zero)
    z_h = jnp.zeros((1, 63), jnp.float32)
    z_v = jnp.zeros((63, 1), jnp.float32)
    pr_h = (jnp.concatenate([is_top, z_h], 0)
            + jnp.concatenate([z_h, is_bot], 0))
    pr_v = (jnp.concatenate([is_lef, z_v], 1)
            + jnp.concatenate([z_v, is_rig], 1))
    dh = jnp.where(pr_h > 0, jnp.inf, dh)
    dv = jnp.where(pr_v > 0, jnp.inf, dv)

    inf_col = jnp.full((64, 1), jnp.inf, jnp.float32)
    inf_row = jnp.full((1, 64), jnp.inf, jnp.float32)
    left = jnp.concatenate([dh, inf_col], axis=1)
    right = jnp.concatenate([dv, inf_row], axis=0)
    return jnp.concatenate([left, right], axis=1)


def _sort_kernel_body(low_ref, high_ref, a_ref, sw_ref, si_ref):
    A = a_ref[:]
    keys = []
    for b in range(2):
        lows = []
        for c in range(3):
            X = low_ref[b, c]
            Y = jnp.dot(jnp.dot(A, X, preferred_element_type=jnp.float32),
                        A.T, preferred_element_type=jnp.float32)
            lows.append(Y)
        low = jnp.stack(lows, 0)
        keys.append(_edge_key_mat(low))
        keys.append(_edge_key_mat(high_ref[b]))
    K = jnp.stack(keys, 0)

    row = lax.broadcasted_iota(jnp.int32, (64, 128), 0)
    col = lax.broadcasted_iota(jnp.int32, (64, 128), 1)
    lin = row * 128 + col
    idx = jnp.where(col < 63, row * 63 + col, 4032 + row * 64 + (col - 64))
    pad = (col == 63) | ((col >= 64) & (row == 63))
    idx = jnp.where(pad, _MP + lin, idx)
    I = jnp.broadcast_to(idx[None], (4, 64, 128)).astype(jnp.int32)
    lin3 = jnp.broadcast_to(lin[None], (4, 64, 128))

    for k in range(13):
        for j in range(k, -1, -1):
            d = 1 << j
            if d < 128:
                ax, sh, size = 2, d, 128
            else:
                ax, sh, size = 1, d >> 7, 64
            low_half = (lin3 & d) == 0
            pK = jnp.where(low_half, pltpu.roll(K, size - sh, ax),
                           pltpu.roll(K, sh, ax))
            pI = jnp.where(low_half, pltpu.roll(I, size - sh, ax),
                           pltpu.roll(I, sh, ax))
            asc = (lin3 & (1 << (k + 1))) == 0
            partner_less = (pK < K) | ((pK == K) & (pI < I))
            take = asc ^ low_half ^ partner_less
            K = jnp.where(take, pK, K)
            I = jnp.where(take, pI, I)

    sw_ref[:] = K
    si_ref[:] = I


def _tc_sort(low_feats, high_feats):
    A = jnp.asarray(_A64)
    sw, si = pl.pallas_call(
        _sort_kernel_body,
        out_shape=(
            jax.ShapeDtypeStruct((4, 64, 128), jnp.float32),
            jax.ShapeDtypeStruct((4, 64, 128), jnp.int32),
        ),
    )(low_feats, high_feats, A)
    return sw.reshape(4 * _MP), si.reshape(4 * _MP)


_IOTA = lambda: lax.iota(jnp.int32, 16)


def _vg(ref, idx, mask=None):
    return plsc.load_gather(ref, [idx], mask=mask)


def _splat(x, dtype=jnp.int32):
    return jnp.full((16,), x, dtype)


def _lane0():
    return _IOTA() == 0


def _sstore(ref, idx_vec, val_vec):
    plsc.store_scatter(ref, [idx_vec], val_vec, mask=_lane0())


def _fill(ref, n_chunks, value, dtype):
    def body(i, _):
        idx = i * 16 + _IOTA()
        plsc.store_scatter(ref, [idx], _splat(value, dtype))
        return 0
    lax.fori_loop(0, n_chunks, body, 0)


def _uf_find(par_ref, x0):
    p0 = _vg(par_ref, x0)

    def cond(c):
        x, p = c
        return jnp.any(p != x)

    def body(c):
        x, p = c
        g = _vg(par_ref, p)
        _sstore(par_ref, x, g)
        return g, _vg(par_ref, g)

    x, _ = lax.while_loop(cond, body, (x0, p0))
    return x


def _build_tree(sw_ref, sidx_ref, par_ref, adjm_ref, wes_ref,
                parent_ref, d2w_ref, ordq_ref, lvl_ref):
    iota = _IOTA()

    def initb(i, _):
        idx = i * 16 + iota
        plsc.store_scatter(par_ref, [idx], idx)
        plsc.store_scatter(adjm_ref, [idx], _splat(0))
        plsc.store_scatter(parent_ref, [idx], _splat(-1))
        plsc.store_scatter(d2w_ref, [idx], _splat(0.0, jnp.float32))
        return 0
    lax.fori_loop(0, _N // 16, initb, 0)

    def bc(vec, j):
        return vec.at[_splat(j)].get(mode="promise_in_bounds")

    def find_loop(x, p):
        def cond(c):
            x, p = c
            return jnp.any(p != x)

        def body(c):
            x, p = c
            live = p != x
            g = _vg(par_ref, p)
            plsc.store_scatter(par_ref, [x], g, mask=live)
            x2 = jnp.where(live, g, x)
            return x2, _vg(par_ref, x2)

        return lax.while_loop(cond, body, (x, p))[0]

    def kr_cond(c):
        e, cnt = c
        return jnp.any((e < _M) & (cnt < _N - 1))

    def kr_body(c):
        e, cnt = c
        lane8 = iota & 7
        eidx8 = _vg(sidx_ref, e + lane8)
        w8 = _vg(sw_ref, e + lane8)
        horiz8 = eidx8 < 4032
        r8 = eidx8 // 63
        c8 = eidx8 - r8 * 63
        a8 = jnp.where(horiz8, r8 * 64 + c8, eidx8 - 4032)
        b8 = jnp.where(horiz8, a8 + 1, a8 + 64)
        x0 = jnp.where(iota < 8, a8, b8)
        roots = find_loop(x0, _vg(par_ref, x0))

        for j in range(8):
            xj = jnp.where(iota < 8, bc(roots, j), bc(roots, j + 8))
            xj = find_loop(xj, _vg(par_ref, xj))
            ra = bc(xj, 0)
            rb = bc(xj, 8)
            take = ra != rb

            @pl.when(jnp.any(take))
            def _(j=j, ra=ra, rb=rb):
                eidx = bc(eidx8, j)
                horiz = eidx < 4032
                a = bc(a8, j)
                b = bc(b8, j)
                _sstore(par_ref, ra, rb)
                bit_a = jnp.where(horiz, 1, 4)
                bit_b = jnp.where(horiz, 2, 8)
                plsc.addupdate_scatter(adjm_ref, [a], bit_a, mask=_lane0())
                plsc.addupdate_scatter(adjm_ref, [b], bit_b, mask=_lane0())
                wpos = jnp.where(horiz, a, a + _N)
                plsc.store_scatter(wes_ref, [wpos], bc(w8, j), mask=_lane0())

            cnt = cnt + jnp.where(take, 1, 0)

        return e + 8, cnt

    lax.while_loop(kr_cond, kr_body, (_splat(0), _splat(0)))

    _sstore(parent_ref, _splat(0), _splat(0))
    _sstore(ordq_ref, _splat(0), _splat(0))
    _sstore(lvl_ref, _splat(0), _splat(0))
    _sstore(lvl_ref, _splat(1), _splat(1))

    def lvl_cond(c):
        s, t, l = c
        return jnp.any((t < _N) & (l < _N + 2))

    def lvl_body(c):
        s, t, l = c

        def ch_cond(cc):
            q, t2 = cc
            return jnp.any(q < t)

        def ch_body(cc):
            q, t2 = cc
            m_in = iota < (t - q)
            v = _vg(ordq_ref, jnp.where(m_in, q + iota, 0))
            am = _vg(adjm_ref, v)
            for bit, dd in ((1, 1), (2, -1), (4, 64), (8, -64)):
                hasd = m_in & ((am & bit) != 0)
                u = v + dd
                uc = jnp.minimum(jnp.maximum(u, 0), _N - 1)
                pu = _vg(parent_ref, uc, mask=hasd)
                newm = hasd & (pu < 0)
                plsc.store_scatter(parent_ref, [uc], v, mask=newm)
                if bit == 1:
                    wpos = v
                elif bit == 2:
                    wpos = uc
                elif bit == 4:
                    wpos = v + _N
                else:
                    wpos = uc + _N
                wv = _vg(wes_ref, wpos, mask=newm)
                plsc.store_scatter(d2w_ref, [uc], wv, mask=newm)
                kc = plsc.cumsum(newm.astype(jnp.int32))
                pos = t2 + kc - 1
                plsc.store_scatter(ordq_ref, [pos], uc, mask=newm)
                t2 = t2 + plsc.all_reduce_population_count(newm)
            return q + 16, t2

        q, t2 = lax.while_loop(ch_cond, ch_body, (s, t))
        _sstore(lvl_ref, l + 1, t2)
        return t, t2, l + 1

    s, t, l = lax.while_loop(lvl_cond, lvl_body,
                             (_splat(0), _splat(1), _splat(1)))
    _sstore(lvl_ref, _splat(_MAXD_SLOT), l - 1)

    def wgtb(i, _):
        idx = i * 16 + iota
        d2v = _vg(d2w_ref, idx)
        plsc.store_scatter(d2w_ref, [idx], jnp.exp(-_SIGMA * d2v))
        return 0
    lax.fori_loop(0, _N // 16, wgtb, 0)
    _sstore(d2w_ref, _splat(0), _splat(0.0, jnp.float32))


def _run_filter(parent_ref, wgt_ref, ordq_ref, lvl_ref, feat_ref,
                aggf_ref, aggn_ref, df_ref, dn_ref):
    iota = _IOTA()
    maxd = _vg(lvl_ref, _splat(_MAXD_SLOT))

    def initb(i, _):
        idx = i * 16 + iota
        plsc.store_scatter(aggf_ref, [idx], _vg(feat_ref, idx))
        plsc.store_scatter(aggn_ref, [idx], _splat(1.0, jnp.float32))
        return 0
    lax.fori_loop(0, _N // 16, initb, 0)

    def up_cond(l):
        return jnp.any(l >= 1)

    def up_body(l):
        s = _vg(lvl_ref, l)
        e = _vg(lvl_ref, l + 1)

        def cc(q):
            return jnp.any(q < e)

        def cb(q):
            m = iota < (e - q)
            v = jnp.where(m, _vg(ordq_ref, q + iota), 0)
            p = _vg(parent_ref, v)
            w = _vg(wgt_ref, v)
            fa = _vg(aggf_ref, v)
            fn = _vg(aggn_ref, v)
            plsc.addupdate_scatter(aggf_ref, [p], w * fa, mask=m)
            plsc.addupdate_scatter(aggn_ref, [p], w * fn, mask=m)
            return q + 16

        lax.while_loop(cc, cb, s)
        return l - 1

    lax.while_loop(up_cond, up_body, maxd)

    def cpb(i, _):
        idx = i * 16 + iota
        plsc.store_scatter(df_ref, [idx], _vg(aggf_ref, idx))
        plsc.store_scatter(dn_ref, [idx], _vg(aggn_ref, idx))
        return 0
    lax.fori_loop(0, _N // 16, cpb, 0)

    def dn_cond(l):
        return jnp.any(l <= maxd)

    def dn_body(l):
        s = _vg(lvl_ref, l)
        e = _vg(lvl_ref, l + 1)

        def cc(q):
            return jnp.any(q < e)

        def cb(q):
            m = iota < (e - q)
            v = jnp.where(m, _vg(ordq_ref, q + iota), 0)
            p = _vg(parent_ref, v)
            w = _vg(wgt_ref, v)
            fa = _vg(aggf_ref, v)
            fn = _vg(aggn_ref, v)
            nf = fa + w * (_vg(df_ref, p) - w * fa)
            nn = fn + w * (_vg(dn_ref, p) - w * fn)
            plsc.store_scatter(df_ref, [v], nf, mask=m)
            plsc.store_scatter(dn_ref, [v], nn, mask=m)
            return q + 16

        lax.while_loop(cc, cb, s)
        return l + 1

    lax.while_loop(dn_cond, dn_body, _splat(1))

    def outb(i, _):
        idx = i * 16 + iota
        plsc.store_scatter(feat_ref, [idx],
                           _vg(df_ref, idx) / _vg(dn_ref, idx))
        return 0
    lax.fori_loop(0, _N // 16, outb, 0)


def _sc_body(sw_hbm, sidx_hbm, preds_hbm, out_hbm,
             sw_v, sidx_v, par_v, adjm_v, wes_v, parent_v, d2w_v,
             ordq_v, lvl_v, aggf_v, aggn_v, df_v, dn_v, feat_v,
             parent_sh, wgt_sh, ordq_sh, lvl_sh):
    cid = lax.axis_index("c")
    sid = lax.axis_index("s")
    t = 2 * cid + sid

    @pl.when(sid < 2)
    def _():
        pltpu.sync_copy(sw_hbm.at[pl.ds(t * _MP, _MP)], sw_v)
        pltpu.sync_copy(sidx_hbm.at[pl.ds(t * _MP, _MP)], sidx_v)
        _build_tree(sw_v, sidx_v, par_v, adjm_v, wes_v,
                    parent_v, d2w_v, ordq_v, lvl_v)

    @pl.when(sid == 1)
    def _():
        pltpu.sync_copy(parent_v, parent_sh)
        pltpu.sync_copy(d2w_v, wgt_sh)
        pltpu.sync_copy(ordq_v, ordq_sh)
        pltpu.sync_copy(lvl_v, lvl_sh)

    @pl.when(sid == 0)
    def _():
        pltpu.sync_copy(preds_hbm.at[pl.ds(cid * _N, _N)], feat_v)
        _run_filter(parent_v, d2w_v, ordq_v, lvl_v, feat_v,
                    aggf_v, aggn_v, df_v, dn_v)

    plsc.subcore_barrier()

    @pl.when(sid == 0)
    def _():
        pltpu.sync_copy(parent_sh, parent_v)
        pltpu.sync_copy(wgt_sh, d2w_v)
        pltpu.sync_copy(ordq_sh, ordq_v)
        pltpu.sync_copy(lvl_sh, lvl_v)
        _run_filter(parent_v, d2w_v, ordq_v, lvl_v, feat_v,
                    aggf_v, aggn_v, df_v, dn_v)
        pltpu.sync_copy(feat_v, out_hbm.at[pl.ds(cid * _N, _N)])


def _sc_filter(sw, sidx, preds_flat):
    mesh = plsc.VectorSubcoreMesh(core_axis_name="c", subcore_axis_name="s",
                                  num_cores=2, num_subcores=16)
    f = pl.kernel(
        _sc_body,
        out_type=jax.ShapeDtypeStruct((2 * _N,), jnp.float32),
        mesh=mesh,
        compiler_params=pltpu.CompilerParams(needs_layout_passes=False),
        scratch_types=[
            pltpu.VMEM((_MP,), jnp.float32),
            pltpu.VMEM((_MP,), jnp.int32),
            pltpu.VMEM((_N,), jnp.int32),
            pltpu.VMEM((_N,), jnp.int32),
            pltpu.VMEM((2 * _N,), jnp.float32),
            pltpu.VMEM((_N,), jnp.int32),
            pltpu.VMEM((_N,), jnp.float32),
            pltpu.VMEM((_LVL_LEN,), jnp.int32),
            pltpu.VMEM((_LVL_LEN,), jnp.int32),
            pltpu.VMEM((_N,), jnp.float32),
            pltpu.VMEM((_N,), jnp.float32),
            pltpu.VMEM((_N,), jnp.float32),
            pltpu.VMEM((_N,), jnp.float32),
            pltpu.VMEM((_N,), jnp.float32),
            pltpu.VMEM_SHARED((_N,), jnp.int32),
            pltpu.VMEM_SHARED((_N,), jnp.float32),
            pltpu.VMEM_SHARED((_LVL_LEN,), jnp.int32),
            pltpu.VMEM_SHARED((_LVL_LEN,), jnp.int32),
        ],
    )
    return f(sw, sidx, preds_flat)


def _loss_body(preds_ref, sam_ref, unl_ref, as_ref, out_ref):
    sam = sam_ref[:]
    sam_bg = 1.0 - 1.0 / (1.0 + jnp.exp(-sam))
    as_c = (sam_bg + as_ref[:]) * 0.5
    unl = unl_ref[:]
    nc = jnp.sum(unl)
    loss = jnp.sum(unl * jnp.abs(preds_ref[:] - as_c))
    loss = jnp.where(nc > 0, loss / nc, loss)
    out_ref[:, :] = jnp.full((1, 1), _WEIGHT * loss, jnp.float32)


def _tc_loss(preds_f, sam_f, unl_f, as_f):
    out = pl.pallas_call(
        _loss_body,
        out_shape=jax.ShapeDtypeStruct((1, 1), jnp.float32),
    )(preds_f, sam_f, unl_f, as_f)
    return out.reshape(())


def kernel(preds, low_feats, high_feats, SAMSegment, unlabeled_ROIs):
    b = preds.shape[0]
    n = _N
    sw, sidx = _tc_sort(low_feats, high_feats)
    as_flat = _sc_filter(sw, sidx, preds.reshape(b * n))
    return _tc_loss(preds.reshape(b, n), SAMSegment.reshape(b, n),
                    unlabeled_ROIs.reshape(b, n), as_flat.reshape(b, n))
